# baseline (speedup 1.0000x reference)
import jax
import jax.numpy as jnp
from jax.experimental import pallas as pl
from jax.experimental.pallas import tpu as pltpu

_N = 10000
_D = 128
_BM = 400


def _diffusion_block(x_ref, L_ref, o_ref):
    o_ref[...] = -jnp.dot(L_ref[...], x_ref[...],
                          preferred_element_type=jnp.float32)


def kernel(t, x, L):
    del t
    n, d = x.shape
    grid = (n // _BM,)
    out = pl.pallas_call(
        _diffusion_block,
        grid=grid,
        in_specs=[
            pl.BlockSpec((n, d), lambda m: (0, 0)),
            pl.BlockSpec((_BM, n), lambda m: (m, 0)),
        ],
        out_specs=pl.BlockSpec((_BM, d), lambda m: (m, 0)),
        out_shape=jax.ShapeDtypeStruct((n, d), jnp.float32),
        compiler_params=pltpu.CompilerParams(
            dimension_semantics=("parallel",),
        ),
    )(x, L)
    return out
